# baseline (speedup 1.0000x reference)
import functools

import jax
import jax.numpy as jnp
from jax import lax
from jax.experimental import pallas as pl
from jax.experimental.pallas import tpu as pltpu
from jax.experimental.pallas import tpu_sc as plsc

DIM = 64
BATCH = 16384

_NC = 2
_NS = 16
_L = 16
_NW = _NC * _NS
_B_PER_W = BATCH // _NW
_CHUNK = 128
_NCHUNK = _B_PER_W // _CHUNK

_mesh = plsc.VectorSubcoreMesh(core_axis_name="c", subcore_axis_name="s")


@functools.partial(
    pl.kernel,
    mesh=_mesh,
    out_type=jax.ShapeDtypeStruct((BATCH, DIM), jnp.float32),
    scratch_types=[
        pltpu.VMEM((_B_PER_W, 2), jnp.int32),
        pltpu.VMEM((_NCHUNK, _CHUNK), jnp.int32),
        pltpu.VMEM((_NCHUNK, _CHUNK, DIM), jnp.float32),
        pltpu.SemaphoreType.DMA,
        pltpu.SemaphoreType.DMA,
    ],
    compiler_params=pltpu.CompilerParams(
        use_tc_tiling_on_sc=False, needs_layout_passes=False
    ),
)
def _embed_lookup(x_hbm, emb_hbm, out_hbm, xv, idx_v, rows_v, gsem, osem):
    wid = lax.axis_index("s") * _NC + lax.axis_index("c")
    base = wid * _B_PER_W

    pltpu.sync_copy(x_hbm.at[pl.ds(base, _B_PER_W)], xv)

    lane = lax.iota(jnp.int32, _L)
    ones = jnp.full((_L,), 1, jnp.int32)

    gathers = []
    for j in range(_NCHUNK):
        for i in range(_CHUNK // _L):
            rows = lane + (j * _CHUNK + i * _L)
            vals = plsc.load_gather(xv, [rows, ones])
            idx_v[j, pl.ds(i * _L, _L)] = lax.div(vals, 90)
        gathers.append(
            pltpu.async_copy(emb_hbm.at[idx_v.at[j]], rows_v.at[j], gsem)
        )

    outs = []
    for j in range(_NCHUNK):
        gathers[j].wait()
        outs.append(
            pltpu.async_copy(
                rows_v.at[j], out_hbm.at[pl.ds(base + j * _CHUNK, _CHUNK)], osem
            )
        )
    for c in outs:
        c.wait()


def kernel(x, emb):
    return _embed_lookup(x.astype(jnp.int32), emb)


# speedup vs baseline: 1.3828x; 1.3828x over previous
import functools

import jax
import jax.numpy as jnp
from jax import lax
from jax.experimental import pallas as pl
from jax.experimental.pallas import tpu as pltpu
from jax.experimental.pallas import tpu_sc as plsc

NBRICK = 100000
DIM = 64
BATCH = 16384

_NC = 2
_NS = 16
_L = 16
_NW = _NC * _NS
_DPW = DIM // _NW
_OCHUNK = 2048
_NOCHUNK = BATCH // _OCHUNK
_VPC = _OCHUNK // _L
_UNROLL = 16

_mesh = plsc.VectorSubcoreMesh(core_axis_name="c", subcore_axis_name="s")


@functools.partial(
    pl.kernel,
    mesh=_mesh,
    out_type=jax.ShapeDtypeStruct((DIM, BATCH), jnp.float32),
    scratch_types=[
        pltpu.VMEM((BATCH,), jnp.int32),
        pltpu.VMEM((NBRICK,), jnp.float32),
        pltpu.VMEM((2, _OCHUNK), jnp.float32),
        pltpu.SemaphoreType.DMA,
        pltpu.SemaphoreType.DMA,
    ],
    compiler_params=pltpu.CompilerParams(
        use_tc_tiling_on_sc=True, needs_layout_passes=False
    ),
)
def _embed_t(x1_hbm, embt_hbm, outt_hbm, idx_v, row_v, ob, rsem, osem):
    wid = lax.axis_index("s") * _NC + lax.axis_index("c")
    d0 = wid * _DPW

    row_copy = pltpu.async_copy(embt_hbm.at[d0], row_v, rsem)

    pltpu.sync_copy(x1_hbm, idx_v)

    def _div_body(k, carry):
        for i in range(_UNROLL):
            off = k * (_UNROLL * _L) + i * _L
            idx_v[pl.ds(off, _L)] = lax.div(idx_v[pl.ds(off, _L)], 90)
        return carry

    lax.fori_loop(0, BATCH // (_UNROLL * _L), _div_body, 0)

    row_copy.wait()

    for p in range(_DPW):
        d = d0 + p
        if p > 0:
            pltpu.sync_copy(embt_hbm.at[d], row_v)
        outs = []
        for c in range(_NOCHUNK):
            bsel = c % 2
            if len(outs) >= 2:
                outs[-2].wait()

            def _gather_body(k, carry, c=c, bsel=bsel):
                for i in range(_UNROLL):
                    rel = k * (_UNROLL * _L) + i * _L
                    iv = idx_v[pl.ds(c * _OCHUNK + rel, _L)]
                    ob[bsel, pl.ds(rel, _L)] = plsc.load_gather(row_v, [iv])
                return carry

            lax.fori_loop(0, _VPC // _UNROLL, _gather_body, 0)
            outs.append(
                pltpu.async_copy(
                    ob.at[bsel],
                    outt_hbm.at[d, pl.ds(c * _OCHUNK, _OCHUNK)],
                    osem,
                )
            )
        for o in outs[-2:]:
            o.wait()


def kernel(x, emb):
    x1 = x[:, 1].astype(jnp.int32)
    out_t = _embed_t(x1, emb.T)
    return out_t.T


# speedup vs baseline: 2.1279x; 1.5388x over previous
import functools

import jax
import jax.numpy as jnp
from jax import lax
from jax.experimental import pallas as pl
from jax.experimental.pallas import tpu as pltpu
from jax.experimental.pallas import tpu_sc as plsc

NBRICK = 100000
DIM = 64
BATCH = 16384

_NC = 2
_NS = 16
_L = 16
_NW = _NC * _NS
_DPW = DIM // _NW
_OCHUNK = 2048
_NOCHUNK = BATCH // _OCHUNK
_VPC = _OCHUNK // _L
_UNROLL = 16

_mesh = plsc.VectorSubcoreMesh(core_axis_name="c", subcore_axis_name="s")


@functools.partial(
    pl.kernel,
    mesh=_mesh,
    out_type=jax.ShapeDtypeStruct((DIM, BATCH), jnp.float32),
    scratch_types=[
        pltpu.VMEM((BATCH,), jnp.int32),
        pltpu.VMEM((NBRICK,), jnp.float32),
        pltpu.VMEM((2, _OCHUNK), jnp.float32),
        pltpu.SemaphoreType.DMA,
        pltpu.SemaphoreType.DMA,
    ],
    compiler_params=pltpu.CompilerParams(
        use_tc_tiling_on_sc=True, needs_layout_passes=False
    ),
)
def _embed_t(x1_hbm, embt_hbm, outt_hbm, idx_v, row_v, ob, rsem, osem):
    wid = lax.axis_index("s") * _NC + lax.axis_index("c")
    d0 = wid * _DPW

    row_copy = pltpu.async_copy(embt_hbm.at[d0], row_v, rsem)

    pltpu.sync_copy(x1_hbm, idx_v)

    rcp = jnp.float32(1.0 / 90.0)

    def _div_body(k, carry):
        for i in range(_UNROLL):
            off = k * (_UNROLL * _L) + i * _L
            v = idx_v[pl.ds(off, _L)]
            q = (v.astype(jnp.float32) * rcp).astype(jnp.int32)
            r = v - q * 90
            idx_v[pl.ds(off, _L)] = lax.select(r >= 90, q + 1, q)
        return carry

    lax.fori_loop(0, BATCH // (_UNROLL * _L), _div_body, 0)

    row_copy.wait()

    for p in range(_DPW):
        d = d0 + p
        if p > 0:
            pltpu.sync_copy(embt_hbm.at[d], row_v)
        outs = []
        for c in range(_NOCHUNK):
            bsel = c % 2
            if len(outs) >= 2:
                outs[-2].wait()

            def _gather_body(k, carry, c=c, bsel=bsel):
                for i in range(_UNROLL):
                    rel = k * (_UNROLL * _L) + i * _L
                    iv = idx_v[pl.ds(c * _OCHUNK + rel, _L)]
                    ob[bsel, pl.ds(rel, _L)] = plsc.load_gather(row_v, [iv])
                return carry

            lax.fori_loop(0, _VPC // _UNROLL, _gather_body, 0)
            outs.append(
                pltpu.async_copy(
                    ob.at[bsel],
                    outt_hbm.at[d, pl.ds(c * _OCHUNK, _OCHUNK)],
                    osem,
                )
            )
        for o in outs[-2:]:
            o.wait()


def kernel(x, emb):
    x1 = x[:, 1].astype(jnp.int32)
    out_t = _embed_t(x1, emb.T)
    return out_t.T


# speedup vs baseline: 2.1358x; 1.0037x over previous
import functools

import jax
import jax.numpy as jnp
from jax import lax
from jax.experimental import pallas as pl
from jax.experimental.pallas import tpu as pltpu
from jax.experimental.pallas import tpu_sc as plsc

NBRICK = 100000
DIM = 64
BATCH = 16384

_NC = 2
_NS = 16
_L = 16
_NW = _NC * _NS
_DPW = DIM // _NW
_OCHUNK = 4096
_NOCHUNK = BATCH // _OCHUNK
_VPC = _OCHUNK // _L
_UNROLL = 32

_mesh = plsc.VectorSubcoreMesh(core_axis_name="c", subcore_axis_name="s")


@functools.partial(
    pl.kernel,
    mesh=_mesh,
    out_type=jax.ShapeDtypeStruct((DIM, BATCH), jnp.float32),
    scratch_types=[
        pltpu.VMEM((BATCH,), jnp.int32),
        pltpu.VMEM((NBRICK,), jnp.float32),
        pltpu.VMEM((2, _OCHUNK), jnp.float32),
        pltpu.SemaphoreType.DMA,
        pltpu.SemaphoreType.DMA,
        pltpu.SemaphoreType.DMA,
    ],
    compiler_params=pltpu.CompilerParams(
        use_tc_tiling_on_sc=True, needs_layout_passes=False
    ),
)
def _embed_t(x1_hbm, embt_hbm, outt_hbm, idx_v, row_v, ob, rsem, osem, xsem):
    wid = lax.axis_index("s") * _NC + lax.axis_index("c")
    d0 = wid * _DPW

    row_copy = pltpu.async_copy(embt_hbm.at[d0], row_v, rsem)

    pltpu.async_copy(x1_hbm, idx_v, xsem).wait()

    rcp = jnp.float32(1.0 / 90.0)

    def _div_body(k, carry):
        for i in range(_UNROLL):
            off = k * (_UNROLL * _L) + i * _L
            v = idx_v[pl.ds(off, _L)]
            q = (v.astype(jnp.float32) * rcp).astype(jnp.int32)
            r = v - q * 90
            idx_v[pl.ds(off, _L)] = lax.select(r >= 90, q + 1, q)
        return carry

    lax.fori_loop(0, BATCH // (_UNROLL * _L), _div_body, 0)

    row_copy.wait()

    for p in range(_DPW):
        d = d0 + p
        if p > 0:
            pltpu.sync_copy(embt_hbm.at[d], row_v)
        outs = []
        for c in range(_NOCHUNK):
            bsel = c % 2
            if len(outs) >= 2:
                outs[-2].wait()

            def _gather_body(k, carry, c=c, bsel=bsel):
                for i in range(_UNROLL):
                    rel = k * (_UNROLL * _L) + i * _L
                    iv = idx_v[pl.ds(c * _OCHUNK + rel, _L)]
                    ob[bsel, pl.ds(rel, _L)] = plsc.load_gather(row_v, [iv])
                return carry

            lax.fori_loop(0, _VPC // _UNROLL, _gather_body, 0)
            outs.append(
                pltpu.async_copy(
                    ob.at[bsel],
                    outt_hbm.at[d, pl.ds(c * _OCHUNK, _OCHUNK)],
                    osem,
                )
            )
        for o in outs[-2:]:
            o.wait()


def kernel(x, emb):
    x1 = x[:, 1].astype(jnp.int32)
    out_t = _embed_t(x1, emb.T)
    return out_t.T
